# Initial kernel scaffold; baseline (speedup 1.0000x reference)
import jax
import jax.numpy as jnp
from jax.experimental import pallas as pl


def kernel(features, edge_index, edge_type, Wq, bq, Wk, bk, Wv, bv, Ws, bs, Wg, bg, W1, b1, W2):
    raise NotImplementedError("write your pallas kernel here")


# speedup vs baseline: 2.9297x; 2.9297x over previous
import jax
import jax.numpy as jnp
from jax.experimental import pallas as pl

_T = 3
_H = 4
_C = 256
_S = 4
_HID = 128
_BLK = 1024


def _proj_kernel(x_ref, wq_ref, bq_ref, wk_ref, bk_ref, wv_ref, bv_ref,
                 ws_ref, bs_ref, q_ref, k_ref, v_ref, s_ref):
    x = x_ref[...]
    q_ref[0, :, :] = jnp.dot(x, wq_ref[0], preferred_element_type=jnp.float32) + bq_ref[0]
    k_ref[0, :, :] = jnp.dot(x, wk_ref[0], preferred_element_type=jnp.float32) + bk_ref[0]
    v_ref[0, :, :] = jnp.dot(x, wv_ref[0], preferred_element_type=jnp.float32) + bv_ref[0]
    s_ref[0, :, :] = jnp.dot(x, ws_ref[0], preferred_element_type=jnp.float32) + bs_ref[0]


def _gate_sem_kernel(u_ref, x_ref, wg_ref, bg_ref, w1_ref, b1_ref, w2_ref,
                     z_ref, w_ref):
    x = x_ref[...]
    in_dim = x.shape[1]
    zs = []
    for t in range(_T):
        u_t = u_ref[:, t * _C:(t + 1) * _C]
        a = jax.nn.sigmoid(
            jnp.dot(u_t, wg_ref[:_C, :], preferred_element_type=jnp.float32)
            + jnp.dot(x, wg_ref[_C:, :], preferred_element_type=jnp.float32)
            + bg_ref[0])
        z_t = jnp.tanh(u_t) * a + x * (1.0 - a)
        zs.append(z_t)
        z_ref[:, t * _C:(t + 1) * _C] = z_t
    cols = []
    for h in range(_S):
        for t in range(_T):
            t1 = jnp.tanh(
                jnp.dot(zs[t], w1_ref[h], preferred_element_type=jnp.float32)
                + b1_ref[h])
            cols.append(jnp.sum(t1 * w2_ref[h], axis=1, keepdims=True))
    w_ref[...] = jnp.concatenate(cols, axis=1)


def _combine_kernel(z_ref, c_ref, o_ref):
    prod = z_ref[...] * c_ref[0]
    o_ref[...] = prod[:, :_C] + prod[:, _C:2 * _C] + prod[:, 2 * _C:]


def kernel(features, edge_index, edge_type, Wq, bq, Wk, bk, Wv, bv, Ws, bs,
           Wg, bg, W1, b1, W2):
    n, in_dim = features.shape
    e = edge_type.shape[0]
    hc = Wq.shape[-1]
    nb = -(-n // _BLK)
    npad = nb * _BLK
    f32 = jnp.float32
    xp = jnp.pad(features, ((0, npad - n), (0, 0)))

    q, k, v, s = pl.pallas_call(
        _proj_kernel,
        grid=(_T, nb),
        in_specs=[
            pl.BlockSpec((_BLK, in_dim), lambda t, i: (i, 0)),
            pl.BlockSpec((1, in_dim, hc), lambda t, i: (t, 0, 0)),
            pl.BlockSpec((1, 1, hc), lambda t, i: (t, 0, 0)),
            pl.BlockSpec((1, in_dim, hc), lambda t, i: (t, 0, 0)),
            pl.BlockSpec((1, 1, hc), lambda t, i: (t, 0, 0)),
            pl.BlockSpec((1, in_dim, hc), lambda t, i: (t, 0, 0)),
            pl.BlockSpec((1, 1, hc), lambda t, i: (t, 0, 0)),
            pl.BlockSpec((1, in_dim, _C), lambda t, i: (t, 0, 0)),
            pl.BlockSpec((1, 1, _C), lambda t, i: (t, 0, 0)),
        ],
        out_specs=[
            pl.BlockSpec((1, _BLK, hc), lambda t, i: (t, i, 0)),
            pl.BlockSpec((1, _BLK, hc), lambda t, i: (t, i, 0)),
            pl.BlockSpec((1, _BLK, hc), lambda t, i: (t, i, 0)),
            pl.BlockSpec((1, _BLK, _C), lambda t, i: (t, i, 0)),
        ],
        out_shape=[
            jax.ShapeDtypeStruct((_T, npad, hc), f32),
            jax.ShapeDtypeStruct((_T, npad, hc), f32),
            jax.ShapeDtypeStruct((_T, npad, hc), f32),
            jax.ShapeDtypeStruct((_T, npad, _C), f32),
        ],
    )(xp, Wq, bq.reshape(_T, 1, hc), Wk, bk.reshape(_T, 1, hc),
      Wv, bv.reshape(_T, 1, hc), Ws, bs.reshape(_T, 1, _C))

    src = edge_index[0]
    dst = edge_index[1]
    et = edge_type
    qd = q[et, dst].reshape(e, _H, _C)
    ks = k[et, src].reshape(e, _H, _C)
    vs = v[et, src].reshape(e, _H, _C)
    alpha = jnp.sum(qd * ks, axis=-1) * (1.0 / 16.0)
    seg = dst * _T + et
    nseg = n * _T
    amax = jax.ops.segment_max(alpha, seg, num_segments=nseg)
    amax = jnp.where(jnp.isfinite(amax), amax, 0.0)
    ex = jnp.exp(alpha - amax[seg])
    den = jax.ops.segment_sum(ex, seg, num_segments=nseg)
    att = ex / (den[seg] + 1e-16)
    m = jax.ops.segment_sum(vs * att[:, :, None], seg, num_segments=nseg)
    u = m.reshape(n, _T, _H, _C).mean(axis=2)
    u = u + s[:, :n].transpose(1, 0, 2)
    up = jnp.pad(u.reshape(n, _T * _C), ((0, npad - n), (0, 0)))

    z, wps = pl.pallas_call(
        _gate_sem_kernel,
        grid=(nb,),
        in_specs=[
            pl.BlockSpec((_BLK, _T * _C), lambda i: (i, 0)),
            pl.BlockSpec((_BLK, in_dim), lambda i: (i, 0)),
            pl.BlockSpec(Wg.shape, lambda i: (0, 0)),
            pl.BlockSpec((1, in_dim), lambda i: (0, 0)),
            pl.BlockSpec((_S, _C, _HID), lambda i: (0, 0, 0)),
            pl.BlockSpec((_S, 1, _HID), lambda i: (0, 0, 0)),
            pl.BlockSpec((_S, 1, _HID), lambda i: (0, 0, 0)),
        ],
        out_specs=[
            pl.BlockSpec((_BLK, _T * _C), lambda i: (i, 0)),
            pl.BlockSpec((_BLK, _S * _T), lambda i: (i, 0)),
        ],
        out_shape=[
            jax.ShapeDtypeStruct((npad, _T * _C), f32),
            jax.ShapeDtypeStruct((npad, _S * _T), f32),
        ],
    )(up, xp, Wg, bg.reshape(1, in_dim), W1, b1.reshape(_S, 1, _HID),
      W2.transpose(0, 2, 1))

    w = wps[:n].mean(axis=0).reshape(_S, _T)
    beta = jax.nn.softmax(w, axis=1)
    coef = beta.mean(axis=0)
    coef_row = jnp.repeat(coef, _C)[None, :]

    out = pl.pallas_call(
        _combine_kernel,
        grid=(nb,),
        in_specs=[
            pl.BlockSpec((_BLK, _T * _C), lambda i: (i, 0)),
            pl.BlockSpec((1, _T * _C), lambda i: (0, 0)),
        ],
        out_specs=pl.BlockSpec((_BLK, _C), lambda i: (i, 0)),
        out_shape=jax.ShapeDtypeStruct((npad, _C), f32),
    )(z, coef_row)
    return out[:n]
